# baseline (speedup 1.0000x reference)
import functools

import jax
import jax.numpy as jnp
from jax import lax
from jax.experimental import pallas as pl
from jax.experimental.pallas import tpu as pltpu
from jax.experimental.pallas import tpu_sc as plsc

VOCAB = 1000000
VEC = 64
HID = 300
NCLS = 100
B = 1024
L = 200
TOK = B * L
NSPLIT = 2
TOKH = TOK // NSPLIT
BH = B // NSPLIT

NC = 2
NS = 16
NW = NC * NS
ROWS_W = TOKH // NW
CHUNK = 400
NCH = ROWS_W // CHUNK

BB = 32
TB = BB * L


HALF = 1 << 19
VB = 16384
NVB = HALF // VB


def _tr_body(tl_ref, tr_ref, out_ref):
    eye = (lax.broadcasted_iota(jnp.int32, (VEC, VEC), 0) ==
           lax.broadcasted_iota(jnp.int32, (VEC, VEC), 1)).astype(jnp.bfloat16)
    ttl = lax.dot_general(
        tl_ref[...].astype(jnp.bfloat16), eye,
        (((0,), (0,)), ((), ())), preferred_element_type=jnp.float32)
    ttr = lax.dot_general(
        tr_ref[...].astype(jnp.bfloat16), eye,
        (((0,), (0,)), ((), ())), preferred_element_type=jnp.float32)
    out_ref[:, :VEC] = ttl
    out_ref[:, VEC:] = ttr


def _transpose_table(table_t):
    return pl.pallas_call(
        _tr_body,
        grid=(NVB,),
        in_specs=[
            pl.BlockSpec((VEC, VB), lambda i: (0, i)),
            pl.BlockSpec(
                (VEC, VB),
                lambda i: (0, jnp.minimum(i + NVB, (VOCAB - 1) // VB))),
        ],
        out_specs=pl.BlockSpec((VB, 2 * VEC), lambda i: (i, 0)),
        out_shape=jax.ShapeDtypeStruct((HALF, 2 * VEC), jnp.float32),
    )(table_t, table_t)


def _sc_gather(idx, table2):
    mesh = plsc.VectorSubcoreMesh(core_axis_name="c", subcore_axis_name="s")

    @functools.partial(
        pl.kernel,
        mesh=mesh,
        out_type=jax.ShapeDtypeStruct((TOKH, 2 * VEC), jnp.float32),
        scratch_types=[
            pltpu.VMEM((ROWS_W,), jnp.int32),
            pltpu.VMEM((CHUNK, 2 * VEC), jnp.float32),
            pltpu.VMEM((CHUNK, 2 * VEC), jnp.float32),
            pltpu.SemaphoreType.DMA,
            pltpu.SemaphoreType.DMA,
        ],
    )
    def gather_kernel(idx_hbm, table_hbm, out_hbm, idx_v, buf0, buf1, sem0, sem1):
        wid = lax.axis_index("s") * NC + lax.axis_index("c")
        base = wid * ROWS_W
        pltpu.sync_copy(idx_hbm.at[pl.ds(base, ROWS_W)], idx_v)
        bufs = (buf0, buf1)
        sems = (sem0, sem1)
        inflight = pltpu.async_copy(
            table_hbm.at[idx_v.at[pl.ds(0, CHUNK)]], bufs[0], sems[0])
        for c in range(NCH):
            nxt = None
            if c + 1 < NCH:
                nxt = pltpu.async_copy(
                    table_hbm.at[idx_v.at[pl.ds((c + 1) * CHUNK, CHUNK)]],
                    bufs[(c + 1) % 2], sems[(c + 1) % 2])
            inflight.wait()
            pltpu.sync_copy(bufs[c % 2],
                            out_hbm.at[pl.ds(base + c * CHUNK, CHUNK)])
            if nxt is not None:
                inflight = nxt

    return gather_kernel(idx, table2)


def _tc_body(g_ref, par_ref, w1d_ref, b1_ref, wc_ref, bc_ref, lab_ref, out_ref):
    i = pl.program_id(0)
    g = g_ref[...]
    par = par_ref[...]
    lane = lax.broadcasted_iota(jnp.int32, (TB, 2 * VEC), 1)
    keep = (lane < VEC) == (par == 0)
    gm = jnp.where(keep, g, 0.0).astype(jnp.bfloat16)
    h = lax.dot_general(
        gm, w1d_ref[...].astype(jnp.bfloat16),
        (((1,), (0,)), ((), ())), preferred_element_type=jnp.float32)
    h = jnp.maximum(h + b1_ref[...], 0.0)
    pre = lax.dot_general(
        h.astype(jnp.bfloat16), wc_ref[...].astype(jnp.bfloat16),
        (((1,), (0,)), ((), ())), preferred_element_type=jnp.float32)
    pre = pre + bc_ref[...]
    pre = jnp.max(pre.reshape(BB, L, NCLS), axis=1)

    m = jnp.max(pre, axis=-1, keepdims=True)
    z = jnp.sum(jnp.exp(pre - m), axis=-1, keepdims=True)
    log_z = m + jnp.log(z)
    onehot = lax.broadcasted_iota(jnp.int32, (BB, NCLS), 1) == lab_ref[...]
    ll = jnp.sum(jnp.where(onehot, pre, 0.0), axis=-1, keepdims=True)
    part = jnp.sum(log_z - ll) * (1.0 / B)

    @pl.when(i == 0)
    def _init():
        out_ref[...] = jnp.zeros((1, 1), jnp.float32)

    out_ref[...] += part


def _tc_loss(gathered, parity, label, W1d, b1, Wc, bc, interpret=False):
    out = pl.pallas_call(
        _tc_body,
        grid=(BH // BB,),
        in_specs=[
            pl.BlockSpec((TB, 2 * VEC), lambda i: (i, 0)),
            pl.BlockSpec((TB, 1), lambda i: (i, 0)),
            pl.BlockSpec((2 * VEC, HID), lambda i: (0, 0)),
            pl.BlockSpec((1, HID), lambda i: (0, 0)),
            pl.BlockSpec((HID, NCLS), lambda i: (0, 0)),
            pl.BlockSpec((1, NCLS), lambda i: (0, 0)),
            pl.BlockSpec((BB, 1), lambda i: (i, 0)),
        ],
        out_specs=pl.BlockSpec((1, 1), lambda i: (0, 0)),
        out_shape=jax.ShapeDtypeStruct((1, 1), jnp.float32),
        interpret=interpret,
    )(gathered, parity, W1d, b1.reshape(1, HID), Wc, bc.reshape(1, NCLS),
      label.reshape(BH, 1).astype(jnp.int32))
    return out[0, 0]


def kernel(x, label, emb_table, W1, b1, Wc, bc):
    xf = x.reshape(TOK).astype(jnp.int32)
    table2 = _transpose_table(emb_table.T)
    idx = xf & (HALF - 1)
    parity = (xf >> 19).reshape(TOK, 1)
    W1d = jnp.concatenate([W1, W1], axis=0)
    loss = jnp.float32(0)
    for q in range(NSPLIT):
        gq = _sc_gather(lax.slice(idx, (q * TOKH,), ((q + 1) * TOKH,)), table2)
        loss = loss + _tc_loss(
            gq, lax.slice(parity, (q * TOKH, 0), ((q + 1) * TOKH, 1)),
            lax.slice(label, (q * BH,), ((q + 1) * BH,)),
            W1d, b1, Wc, bc)
    return loss


# speedup vs baseline: 1.1857x; 1.1857x over previous
import functools

import jax
import jax.numpy as jnp
from jax import lax
from jax.experimental import pallas as pl
from jax.experimental.pallas import tpu as pltpu
from jax.experimental.pallas import tpu_sc as plsc

VOCAB = 1000000
VEC = 64
HID = 300
NCLS = 100
B = 1024
L = 200
TOK = B * L
NSPLIT = 2
TOKH = TOK // NSPLIT
BH = B // NSPLIT

NC = 2
NS = 16
NW = NC * NS
ROWS_W = TOKH // NW
CHUNK = 400
NCH = ROWS_W // CHUNK

BB = 32
TB = BB * L


VB = 16384
NVB = -(-VOCAB // VB)


def _tr_body(t_ref, out_ref):
    eye = (lax.broadcasted_iota(jnp.int32, (VEC, VEC), 0) ==
           lax.broadcasted_iota(jnp.int32, (VEC, VEC), 1)).astype(jnp.bfloat16)
    tt = lax.dot_general(
        t_ref[...].astype(jnp.bfloat16), eye,
        (((0,), (0,)), ((), ())), preferred_element_type=jnp.float32)
    out_ref[:, :VEC] = tt


def _transpose_table(table_t):
    return pl.pallas_call(
        _tr_body,
        grid=(NVB,),
        in_specs=[pl.BlockSpec((VEC, VB), lambda i: (0, i))],
        out_specs=pl.BlockSpec((VB, 2 * VEC), lambda i: (i, 0)),
        out_shape=jax.ShapeDtypeStruct((VOCAB, 2 * VEC), jnp.float32),
    )(table_t)


def _sc_gather(idx, table2):
    mesh = plsc.VectorSubcoreMesh(core_axis_name="c", subcore_axis_name="s")

    @functools.partial(
        pl.kernel,
        mesh=mesh,
        out_type=jax.ShapeDtypeStruct((TOKH, 2 * VEC), jnp.float32),
        scratch_types=[
            pltpu.VMEM((ROWS_W,), jnp.int32),
            pltpu.VMEM((CHUNK, 2 * VEC), jnp.float32),
            pltpu.VMEM((CHUNK, 2 * VEC), jnp.float32),
            pltpu.SemaphoreType.DMA,
            pltpu.SemaphoreType.DMA,
        ],
    )
    def gather_kernel(idx_hbm, table_hbm, out_hbm, idx_v, buf0, buf1, sem0, sem1):
        wid = lax.axis_index("s") * NC + lax.axis_index("c")
        base = wid * ROWS_W
        pltpu.sync_copy(idx_hbm.at[pl.ds(base, ROWS_W)], idx_v)
        bufs = (buf0, buf1)
        sems = (sem0, sem1)
        inflight = pltpu.async_copy(
            table_hbm.at[idx_v.at[pl.ds(0, CHUNK)]], bufs[0], sems[0])
        for c in range(NCH):
            nxt = None
            if c + 1 < NCH:
                nxt = pltpu.async_copy(
                    table_hbm.at[idx_v.at[pl.ds((c + 1) * CHUNK, CHUNK)]],
                    bufs[(c + 1) % 2], sems[(c + 1) % 2])
            inflight.wait()
            pltpu.sync_copy(bufs[c % 2],
                            out_hbm.at[pl.ds(base + c * CHUNK, CHUNK)])
            if nxt is not None:
                inflight = nxt

    return gather_kernel(idx, table2)


def _tc_body(g_ref, w1d_ref, b1_ref, wc_ref, bc_ref, lab_ref, out_ref):
    i = pl.program_id(0)
    g = g_ref[...]
    lane = lax.broadcasted_iota(jnp.int32, (TB, 2 * VEC), 1)
    gm = jnp.where(lane < VEC, g, 0.0).astype(jnp.bfloat16)
    h = lax.dot_general(
        gm, w1d_ref[...].astype(jnp.bfloat16),
        (((1,), (0,)), ((), ())), preferred_element_type=jnp.float32)
    h = jnp.maximum(h + b1_ref[...], 0.0)
    pre = lax.dot_general(
        h.astype(jnp.bfloat16), wc_ref[...].astype(jnp.bfloat16),
        (((1,), (0,)), ((), ())), preferred_element_type=jnp.float32)
    pre = pre + bc_ref[...]
    pre = jnp.max(pre.reshape(BB, L, NCLS), axis=1)

    m = jnp.max(pre, axis=-1, keepdims=True)
    z = jnp.sum(jnp.exp(pre - m), axis=-1, keepdims=True)
    log_z = m + jnp.log(z)
    onehot = lax.broadcasted_iota(jnp.int32, (BB, NCLS), 1) == lab_ref[...]
    ll = jnp.sum(jnp.where(onehot, pre, 0.0), axis=-1, keepdims=True)
    part = jnp.sum(log_z - ll) * (1.0 / B)

    @pl.when(i == 0)
    def _init():
        out_ref[...] = jnp.zeros((1, 1), jnp.float32)

    out_ref[...] += part


def _tc_loss(gathered, label, W1d, b1, Wc, bc, interpret=False):
    out = pl.pallas_call(
        _tc_body,
        grid=(BH // BB,),
        in_specs=[
            pl.BlockSpec((TB, 2 * VEC), lambda i: (i, 0)),
            pl.BlockSpec((2 * VEC, HID), lambda i: (0, 0)),
            pl.BlockSpec((1, HID), lambda i: (0, 0)),
            pl.BlockSpec((HID, NCLS), lambda i: (0, 0)),
            pl.BlockSpec((1, NCLS), lambda i: (0, 0)),
            pl.BlockSpec((BB, 1), lambda i: (i, 0)),
        ],
        out_specs=pl.BlockSpec((1, 1), lambda i: (0, 0)),
        out_shape=jax.ShapeDtypeStruct((1, 1), jnp.float32),
        interpret=interpret,
    )(gathered, W1d, b1.reshape(1, HID), Wc, bc.reshape(1, NCLS),
      label.reshape(BH, 1).astype(jnp.int32))
    return out[0, 0]


def kernel(x, label, emb_table, W1, b1, Wc, bc):
    xf = x.reshape(TOK).astype(jnp.int32)
    table2 = _transpose_table(emb_table.T)
    W1d = jnp.concatenate([W1, W1], axis=0)
    loss = jnp.float32(0)
    for q in range(NSPLIT):
        gq = _sc_gather(lax.slice(xf, (q * TOKH,), ((q + 1) * TOKH,)), table2)
        loss = loss + _tc_loss(
            gq, lax.slice(label, (q * BH,), ((q + 1) * BH,)),
            W1d, b1, Wc, bc)
    return loss


# speedup vs baseline: 1.2041x; 1.0155x over previous
import functools

import jax
import jax.numpy as jnp
from jax import lax
from jax.experimental import pallas as pl
from jax.experimental.pallas import tpu as pltpu
from jax.experimental.pallas import tpu_sc as plsc

VOCAB = 1000000
VEC = 64
HID = 300
NCLS = 100
B = 1024
L = 200
TOK = B * L
NSPLIT = 4
TOKH = TOK // NSPLIT
BH = B // NSPLIT

NC = 2
NS = 16
NW = NC * NS
ROWS_W = TOKH // NW
CHUNK = 400
NCH = ROWS_W // CHUNK

BB = 32
TB = BB * L


VB = 16384
NVB = -(-VOCAB // VB)


def _tr_body(t_ref, out_ref):
    eye = (lax.broadcasted_iota(jnp.int32, (VEC, VEC), 0) ==
           lax.broadcasted_iota(jnp.int32, (VEC, VEC), 1)).astype(jnp.bfloat16)
    tt = lax.dot_general(
        t_ref[...].astype(jnp.bfloat16), eye,
        (((0,), (0,)), ((), ())), preferred_element_type=jnp.float32)
    out_ref[:, :VEC] = tt


def _transpose_table(table_t):
    return pl.pallas_call(
        _tr_body,
        grid=(NVB,),
        in_specs=[pl.BlockSpec((VEC, VB), lambda i: (0, i))],
        out_specs=pl.BlockSpec((VB, 2 * VEC), lambda i: (i, 0)),
        out_shape=jax.ShapeDtypeStruct((VOCAB, 2 * VEC), jnp.float32),
    )(table_t)


def _sc_gather(idx, table2):
    mesh = plsc.VectorSubcoreMesh(core_axis_name="c", subcore_axis_name="s")

    @functools.partial(
        pl.kernel,
        mesh=mesh,
        out_type=jax.ShapeDtypeStruct((TOKH, 2 * VEC), jnp.float32),
        scratch_types=[
            pltpu.VMEM((ROWS_W,), jnp.int32),
            pltpu.VMEM((CHUNK, 2 * VEC), jnp.float32),
            pltpu.VMEM((CHUNK, 2 * VEC), jnp.float32),
            pltpu.SemaphoreType.DMA,
            pltpu.SemaphoreType.DMA,
        ],
    )
    def gather_kernel(idx_hbm, table_hbm, out_hbm, idx_v, buf0, buf1, sem0, sem1):
        wid = lax.axis_index("s") * NC + lax.axis_index("c")
        base = wid * ROWS_W
        pltpu.sync_copy(idx_hbm.at[pl.ds(base, ROWS_W)], idx_v)
        bufs = (buf0, buf1)
        sems = (sem0, sem1)
        inflight = pltpu.async_copy(
            table_hbm.at[idx_v.at[pl.ds(0, CHUNK)]], bufs[0], sems[0])
        for c in range(NCH):
            nxt = None
            if c + 1 < NCH:
                nxt = pltpu.async_copy(
                    table_hbm.at[idx_v.at[pl.ds((c + 1) * CHUNK, CHUNK)]],
                    bufs[(c + 1) % 2], sems[(c + 1) % 2])
            inflight.wait()
            pltpu.sync_copy(bufs[c % 2],
                            out_hbm.at[pl.ds(base + c * CHUNK, CHUNK)])
            if nxt is not None:
                inflight = nxt

    return gather_kernel(idx, table2)


def _tc_body(g_ref, w1d_ref, b1_ref, wc_ref, bc_ref, lab_ref, out_ref):
    i = pl.program_id(0)
    g = g_ref[...]
    lane = lax.broadcasted_iota(jnp.int32, (TB, 2 * VEC), 1)
    gm = jnp.where(lane < VEC, g, 0.0).astype(jnp.bfloat16)
    h = lax.dot_general(
        gm, w1d_ref[...].astype(jnp.bfloat16),
        (((1,), (0,)), ((), ())), preferred_element_type=jnp.float32)
    h = jnp.maximum(h + b1_ref[...], 0.0)
    pre = lax.dot_general(
        h.astype(jnp.bfloat16), wc_ref[...].astype(jnp.bfloat16),
        (((1,), (0,)), ((), ())), preferred_element_type=jnp.float32)
    pre = pre + bc_ref[...]
    pre = jnp.max(pre.reshape(BB, L, NCLS), axis=1)

    m = jnp.max(pre, axis=-1, keepdims=True)
    z = jnp.sum(jnp.exp(pre - m), axis=-1, keepdims=True)
    log_z = m + jnp.log(z)
    onehot = lax.broadcasted_iota(jnp.int32, (BB, NCLS), 1) == lab_ref[...]
    ll = jnp.sum(jnp.where(onehot, pre, 0.0), axis=-1, keepdims=True)
    part = jnp.sum(log_z - ll) * (1.0 / B)

    @pl.when(i == 0)
    def _init():
        out_ref[...] = jnp.zeros((1, 1), jnp.float32)

    out_ref[...] += part


def _tc_loss(gathered, label, W1d, b1, Wc, bc, interpret=False):
    out = pl.pallas_call(
        _tc_body,
        grid=(BH // BB,),
        in_specs=[
            pl.BlockSpec((TB, 2 * VEC), lambda i: (i, 0)),
            pl.BlockSpec((2 * VEC, HID), lambda i: (0, 0)),
            pl.BlockSpec((1, HID), lambda i: (0, 0)),
            pl.BlockSpec((HID, NCLS), lambda i: (0, 0)),
            pl.BlockSpec((1, NCLS), lambda i: (0, 0)),
            pl.BlockSpec((BB, 1), lambda i: (i, 0)),
        ],
        out_specs=pl.BlockSpec((1, 1), lambda i: (0, 0)),
        out_shape=jax.ShapeDtypeStruct((1, 1), jnp.float32),
        interpret=interpret,
    )(gathered, W1d, b1.reshape(1, HID), Wc, bc.reshape(1, NCLS),
      label.reshape(BH, 1).astype(jnp.int32))
    return out[0, 0]


def kernel(x, label, emb_table, W1, b1, Wc, bc):
    xf = x.reshape(TOK).astype(jnp.int32)
    table2 = _transpose_table(emb_table.T)
    W1d = jnp.concatenate([W1, W1], axis=0)
    loss = jnp.float32(0)
    for q in range(NSPLIT):
        gq = _sc_gather(lax.slice(xf, (q * TOKH,), ((q + 1) * TOKH,)), table2)
        loss = loss + _tc_loss(
            gq, lax.slice(label, (q * BH,), ((q + 1) * BH,)),
            W1d, b1, Wc, bc)
    return loss


# speedup vs baseline: 1.2076x; 1.0029x over previous
import functools

import jax
import jax.numpy as jnp
from jax import lax
from jax.experimental import pallas as pl
from jax.experimental.pallas import tpu as pltpu
from jax.experimental.pallas import tpu_sc as plsc

VOCAB = 1000000
VEC = 64
HID = 300
NCLS = 100
B = 1024
L = 200
TOK = B * L
NSPLIT = 8
TOKH = TOK // NSPLIT
BH = B // NSPLIT

NC = 2
NS = 16
NW = NC * NS
ROWS_W = TOKH // NW
CHUNK = 400
NCH = ROWS_W // CHUNK

BB = 32
TB = BB * L


VB = 32768
NVB = -(-VOCAB // VB)


def _tr_body(t_ref, out_ref):
    eye = (lax.broadcasted_iota(jnp.int32, (VEC, VEC), 0) ==
           lax.broadcasted_iota(jnp.int32, (VEC, VEC), 1)).astype(jnp.bfloat16)
    tt = lax.dot_general(
        t_ref[...].astype(jnp.bfloat16), eye,
        (((0,), (0,)), ((), ())), preferred_element_type=jnp.float32)
    out_ref[:, :VEC] = tt


def _transpose_table(table_t):
    return pl.pallas_call(
        _tr_body,
        grid=(NVB,),
        in_specs=[pl.BlockSpec((VEC, VB), lambda i: (0, i))],
        out_specs=pl.BlockSpec((VB, 2 * VEC), lambda i: (i, 0)),
        out_shape=jax.ShapeDtypeStruct((VOCAB, 2 * VEC), jnp.float32),
    )(table_t)


def _sc_gather(idx, table2):
    mesh = plsc.VectorSubcoreMesh(core_axis_name="c", subcore_axis_name="s")

    @functools.partial(
        pl.kernel,
        mesh=mesh,
        out_type=jax.ShapeDtypeStruct((TOKH, 2 * VEC), jnp.float32),
        scratch_types=[
            pltpu.VMEM((ROWS_W,), jnp.int32),
            pltpu.VMEM((CHUNK, 2 * VEC), jnp.float32),
            pltpu.VMEM((CHUNK, 2 * VEC), jnp.float32),
            pltpu.SemaphoreType.DMA,
            pltpu.SemaphoreType.DMA,
        ],
    )
    def gather_kernel(idx_hbm, table_hbm, out_hbm, idx_v, buf0, buf1, sem0, sem1):
        wid = lax.axis_index("s") * NC + lax.axis_index("c")
        base = wid * ROWS_W
        pltpu.sync_copy(idx_hbm.at[pl.ds(base, ROWS_W)], idx_v)
        bufs = (buf0, buf1)
        sems = (sem0, sem1)
        inflight = pltpu.async_copy(
            table_hbm.at[idx_v.at[pl.ds(0, CHUNK)]], bufs[0], sems[0])
        for c in range(NCH):
            nxt = None
            if c + 1 < NCH:
                nxt = pltpu.async_copy(
                    table_hbm.at[idx_v.at[pl.ds((c + 1) * CHUNK, CHUNK)]],
                    bufs[(c + 1) % 2], sems[(c + 1) % 2])
            inflight.wait()
            pltpu.sync_copy(bufs[c % 2],
                            out_hbm.at[pl.ds(base + c * CHUNK, CHUNK)])
            if nxt is not None:
                inflight = nxt

    return gather_kernel(idx, table2)


def _tc_body(g_ref, w1d_ref, b1_ref, wc_ref, bc_ref, lab_ref, out_ref):
    i = pl.program_id(0)
    g = g_ref[...]
    lane = lax.broadcasted_iota(jnp.int32, (TB, 2 * VEC), 1)
    gm = jnp.where(lane < VEC, g, 0.0).astype(jnp.bfloat16)
    h = lax.dot_general(
        gm, w1d_ref[...].astype(jnp.bfloat16),
        (((1,), (0,)), ((), ())), preferred_element_type=jnp.float32)
    h = jnp.maximum(h + b1_ref[...], 0.0)
    pre = lax.dot_general(
        h.astype(jnp.bfloat16), wc_ref[...].astype(jnp.bfloat16),
        (((1,), (0,)), ((), ())), preferred_element_type=jnp.float32)
    pre = pre + bc_ref[...]
    pre = jnp.max(pre.reshape(BB, L, NCLS), axis=1)

    m = jnp.max(pre, axis=-1, keepdims=True)
    z = jnp.sum(jnp.exp(pre - m), axis=-1, keepdims=True)
    log_z = m + jnp.log(z)
    onehot = lax.broadcasted_iota(jnp.int32, (BB, NCLS), 1) == lab_ref[...]
    ll = jnp.sum(jnp.where(onehot, pre, 0.0), axis=-1, keepdims=True)
    part = jnp.sum(log_z - ll) * (1.0 / B)

    @pl.when(i == 0)
    def _init():
        out_ref[...] = jnp.zeros((1, 1), jnp.float32)

    out_ref[...] += part


def _tc_loss(gathered, label, W1d, b1, Wc, bc, interpret=False):
    out = pl.pallas_call(
        _tc_body,
        grid=(BH // BB,),
        in_specs=[
            pl.BlockSpec((TB, 2 * VEC), lambda i: (i, 0)),
            pl.BlockSpec((2 * VEC, HID), lambda i: (0, 0)),
            pl.BlockSpec((1, HID), lambda i: (0, 0)),
            pl.BlockSpec((HID, NCLS), lambda i: (0, 0)),
            pl.BlockSpec((1, NCLS), lambda i: (0, 0)),
            pl.BlockSpec((BB, 1), lambda i: (i, 0)),
        ],
        out_specs=pl.BlockSpec((1, 1), lambda i: (0, 0)),
        out_shape=jax.ShapeDtypeStruct((1, 1), jnp.float32),
        interpret=interpret,
    )(gathered, W1d, b1.reshape(1, HID), Wc, bc.reshape(1, NCLS),
      label.reshape(BH, 1).astype(jnp.int32))
    return out[0, 0]


def kernel(x, label, emb_table, W1, b1, Wc, bc):
    xf = x.reshape(TOK).astype(jnp.int32)
    table2 = _transpose_table(emb_table.T)
    W1d = jnp.concatenate([W1, W1], axis=0)
    loss = jnp.float32(0)
    for q in range(NSPLIT):
        gq = _sc_gather(lax.slice(xf, (q * TOKH,), ((q + 1) * TOKH,)), table2)
        loss = loss + _tc_loss(
            gq, lax.slice(label, (q * BH,), ((q + 1) * BH,)),
            W1d, b1, Wc, bc)
    return loss


# speedup vs baseline: 1.2939x; 1.0715x over previous
import functools

import jax
import jax.numpy as jnp
from jax import lax
from jax.experimental import pallas as pl
from jax.experimental.pallas import tpu as pltpu
from jax.experimental.pallas import tpu_sc as plsc

VOCAB = 1000000
VEC = 64
HID = 300
NCLS = 100
B = 1024
L = 200
TOK = B * L
NSPLIT = 8
TOKH = TOK // NSPLIT
BH = B // NSPLIT

NC = 2
NS = 16
NW = NC * NS
ROWS_W = TOKH // NW
CHUNK = 400
NCH = ROWS_W // CHUNK

BB = 32
TB = BB * L


HALF = 1 << 19
VB = 16384
NVB = HALF // VB


def _tr_body(tl_ref, tr_ref, out_ref):
    eye = (lax.broadcasted_iota(jnp.int32, (VEC, VEC), 0) ==
           lax.broadcasted_iota(jnp.int32, (VEC, VEC), 1)).astype(jnp.bfloat16)
    ttl = lax.dot_general(
        tl_ref[...].astype(jnp.bfloat16), eye,
        (((0,), (0,)), ((), ())), preferred_element_type=jnp.float32)
    ttr = lax.dot_general(
        tr_ref[...].astype(jnp.bfloat16), eye,
        (((0,), (0,)), ((), ())), preferred_element_type=jnp.float32)
    out_ref[:, :VEC] = ttl
    out_ref[:, VEC:] = ttr


def _transpose_table(table_t):
    return pl.pallas_call(
        _tr_body,
        grid=(NVB,),
        in_specs=[
            pl.BlockSpec((VEC, VB), lambda i: (0, i)),
            pl.BlockSpec(
                (VEC, VB),
                lambda i: (0, jnp.minimum(i + NVB, (VOCAB - 1) // VB))),
        ],
        out_specs=pl.BlockSpec((VB, 2 * VEC), lambda i: (i, 0)),
        out_shape=jax.ShapeDtypeStruct((HALF, 2 * VEC), jnp.float32),
    )(table_t, table_t)


def _sc_gather(idx, table2):
    mesh = plsc.VectorSubcoreMesh(core_axis_name="c", subcore_axis_name="s")

    @functools.partial(
        pl.kernel,
        mesh=mesh,
        out_type=jax.ShapeDtypeStruct((TOKH, 2 * VEC), jnp.float32),
        scratch_types=[
            pltpu.VMEM((ROWS_W,), jnp.int32),
            pltpu.VMEM((CHUNK, 2 * VEC), jnp.float32),
            pltpu.VMEM((CHUNK, 2 * VEC), jnp.float32),
            pltpu.SemaphoreType.DMA,
            pltpu.SemaphoreType.DMA,
        ],
    )
    def gather_kernel(idx_hbm, table_hbm, out_hbm, idx_v, buf0, buf1, sem0, sem1):
        wid = lax.axis_index("s") * NC + lax.axis_index("c")
        base = wid * ROWS_W
        pltpu.sync_copy(idx_hbm.at[pl.ds(base, ROWS_W)], idx_v)
        bufs = (buf0, buf1)
        sems = (sem0, sem1)
        inflight = pltpu.async_copy(
            table_hbm.at[idx_v.at[pl.ds(0, CHUNK)]], bufs[0], sems[0])
        for c in range(NCH):
            nxt = None
            if c + 1 < NCH:
                nxt = pltpu.async_copy(
                    table_hbm.at[idx_v.at[pl.ds((c + 1) * CHUNK, CHUNK)]],
                    bufs[(c + 1) % 2], sems[(c + 1) % 2])
            inflight.wait()
            pltpu.sync_copy(bufs[c % 2],
                            out_hbm.at[pl.ds(base + c * CHUNK, CHUNK)])
            if nxt is not None:
                inflight = nxt

    return gather_kernel(idx, table2)


def _tc_body(g_ref, x_ref, w1d_ref, b1_ref, wc_ref, bc_ref, lab_ref, out_ref):
    i = pl.program_id(0)
    g3 = g_ref[...].reshape(BB, L, 2 * VEC)
    par = (x_ref[...] >> 19)[:, :, None]
    lane = lax.broadcasted_iota(jnp.int32, (BB, L, 2 * VEC), 2)
    keep = (lane < VEC) == (par == 0)
    gm = jnp.where(keep, g3, 0.0).reshape(TB, 2 * VEC).astype(jnp.bfloat16)
    h = lax.dot_general(
        gm, w1d_ref[...].astype(jnp.bfloat16),
        (((1,), (0,)), ((), ())), preferred_element_type=jnp.float32)
    h = jnp.maximum(h + b1_ref[...], 0.0)
    pre = lax.dot_general(
        h.astype(jnp.bfloat16), wc_ref[...].astype(jnp.bfloat16),
        (((1,), (0,)), ((), ())), preferred_element_type=jnp.float32)
    pre = pre + bc_ref[...]
    pre = jnp.max(pre.reshape(BB, L, NCLS), axis=1)

    m = jnp.max(pre, axis=-1, keepdims=True)
    z = jnp.sum(jnp.exp(pre - m), axis=-1, keepdims=True)
    log_z = m + jnp.log(z)
    onehot = lax.broadcasted_iota(jnp.int32, (BB, NCLS), 1) == lab_ref[...]
    ll = jnp.sum(jnp.where(onehot, pre, 0.0), axis=-1, keepdims=True)
    part = jnp.sum(log_z - ll) * (1.0 / B)

    @pl.when(i == 0)
    def _init():
        out_ref[...] = jnp.zeros((1, 1), jnp.float32)

    out_ref[...] += part


def _tc_loss(gathered, xq, label, W1d, b1, Wc, bc, interpret=False):
    out = pl.pallas_call(
        _tc_body,
        grid=(BH // BB,),
        in_specs=[
            pl.BlockSpec((TB, 2 * VEC), lambda i: (i, 0)),
            pl.BlockSpec((BB, L), lambda i: (i, 0)),
            pl.BlockSpec((2 * VEC, HID), lambda i: (0, 0)),
            pl.BlockSpec((1, HID), lambda i: (0, 0)),
            pl.BlockSpec((HID, NCLS), lambda i: (0, 0)),
            pl.BlockSpec((1, NCLS), lambda i: (0, 0)),
            pl.BlockSpec((BB, 1), lambda i: (i, 0)),
        ],
        out_specs=pl.BlockSpec((1, 1), lambda i: (0, 0)),
        out_shape=jax.ShapeDtypeStruct((1, 1), jnp.float32),
        interpret=interpret,
    )(gathered, xq, W1d, b1.reshape(1, HID), Wc, bc.reshape(1, NCLS),
      label.reshape(BH, 1).astype(jnp.int32))
    return out[0, 0]


def kernel(x, label, emb_table, W1, b1, Wc, bc):
    xi = x.astype(jnp.int32)
    xf = xi.reshape(TOK)
    table2 = _transpose_table(emb_table.T)
    idx = xf & (HALF - 1)
    W1d = jnp.concatenate([W1, W1], axis=0)
    loss = jnp.float32(0)
    for q in range(NSPLIT):
        gq = _sc_gather(lax.slice(idx, (q * TOKH,), ((q + 1) * TOKH,)), table2)
        loss = loss + _tc_loss(
            gq, lax.slice(xi, (q * BH, 0), ((q + 1) * BH, L)),
            lax.slice(label, (q * BH,), ((q + 1) * BH,)),
            W1d, b1, Wc, bc)
    return loss
